# Initial kernel scaffold; baseline (speedup 1.0000x reference)
import jax
import jax.numpy as jnp
from jax.experimental import pallas as pl


def kernel(x, edge_index, W1, b1, W2, b2):
    raise NotImplementedError("write your pallas kernel here")


# speedup vs baseline: 2.4601x; 2.4601x over previous
import functools

import jax
import jax.numpy as jnp
from jax.experimental import pallas as pl
from jax.experimental.pallas import tpu as pltpu

_N, _E, _D = 10000, 160000, 256
_BN = 400


def _prep_kernel(x_ref, w_ref, dis_ref, y_ref):
    xw = jnp.dot(x_ref[...], w_ref[...], preferred_element_type=jnp.float32)
    y_ref[...] = xw * dis_ref[...]


def _prep(x, w, dis):
    return pl.pallas_call(
        _prep_kernel,
        grid=(_N // _BN,),
        in_specs=[
            pl.BlockSpec((_BN, _D), lambda i: (i, 0)),
            pl.BlockSpec((_D, _D), lambda i: (0, 0)),
            pl.BlockSpec((_BN, 1), lambda i: (i, 0)),
        ],
        out_specs=pl.BlockSpec((_BN, _D), lambda i: (i, 0)),
        out_shape=jax.ShapeDtypeStruct((_N, _D), jnp.float32),
    )(x, w, dis)


def _finalize_kernel(acc_ref, y_ref, dis_ref, b_ref, o_ref):
    o_ref[...] = jnp.maximum(
        dis_ref[...] * (acc_ref[...] + y_ref[...]) + b_ref[...], 0.0)


def _finalize(acc, y, dis, b):
    return pl.pallas_call(
        _finalize_kernel,
        grid=(_N // _BN,),
        in_specs=[
            pl.BlockSpec((_BN, _D), lambda i: (i, 0)),
            pl.BlockSpec((_BN, _D), lambda i: (i, 0)),
            pl.BlockSpec((_BN, 1), lambda i: (i, 0)),
            pl.BlockSpec((1, _D), lambda i: (0, 0)),
        ],
        out_specs=pl.BlockSpec((_BN, _D), lambda i: (i, 0)),
        out_shape=jax.ShapeDtypeStruct((_N, _D), jnp.float32),
    )(acc, y, dis, b.reshape(1, _D))


def kernel(x, edge_index, W1, b1, W2, b2):
    src = edge_index[0]
    dst = edge_index[1]
    deg = jax.ops.segment_sum(jnp.ones((_E,), jnp.float32), dst,
                              num_segments=_N) + 1.0
    dis = jax.lax.rsqrt(deg).reshape(_N, 1)

    y1 = _prep(x, W1, dis)
    acc1 = jax.ops.segment_sum(y1[src], dst, num_segments=_N)
    h1 = _finalize(acc1, y1, dis, b1)

    y2 = _prep(h1, W2, dis)
    acc2 = jax.ops.segment_sum(y2[src], dst, num_segments=_N)
    return _finalize(acc2, y2, dis, b2)


# speedup vs baseline: 7.4912x; 3.0450x over previous
import functools

import jax
import jax.numpy as jnp
from jax import lax
from jax.experimental import pallas as pl
from jax.experimental.pallas import tpu as pltpu
from jax.experimental.pallas import tpu_sc as plsc

_N, _E, _D = 10000, 160000, 256
_BN = 400

_NC, _NS, _L = 2, 16, 16

_sc_mesh = plsc.VectorSubcoreMesh(core_axis_name="c", subcore_axis_name="s")

_EPT = _E // _NS
_DEGB = 2000
_DEGNB = _EPT // _DEGB


@functools.partial(
    pl.kernel,
    out_type=jax.ShapeDtypeStruct((_N, _L), jnp.float32),
    mesh=_sc_mesh,
    scratch_types=[
        pltpu.VMEM((_DEGB,), jnp.int32),
        pltpu.VMEM((_DEGB, _L), jnp.float32),
        pltpu.VMEM((640, _L), jnp.float32),
        pltpu.VMEM_SHARED((10240, _L), jnp.float32),
    ],
    compiler_params=pltpu.CompilerParams(use_tc_tiling_on_sc=False),
)
def _deg_sc(dst_hbm, deg_hbm, idx_v, ones_v, zeros_v, acc_sh):
    c = lax.axis_index("c")
    s = lax.axis_index("s")

    def fill_ones(i, _):
        ones_v[i] = jnp.ones((_L,), jnp.float32)
        return 0

    lax.fori_loop(0, _DEGB, fill_ones, 0)

    def fill_zeros(i, _):
        zeros_v[i] = jnp.zeros((_L,), jnp.float32)
        return 0

    lax.fori_loop(0, 640, fill_zeros, 0)

    pltpu.sync_copy(zeros_v, acc_sh.at[pl.ds(s * 640, 640)])
    plsc.subcore_barrier()

    def batch(b, _):
        pltpu.sync_copy(dst_hbm.at[pl.ds(s * _EPT + b * _DEGB, _DEGB)], idx_v)
        pltpu.sync_copy(ones_v, acc_sh.at[idx_v], add=True)
        return 0

    lax.fori_loop(0, _DEGNB, batch, 0)
    plsc.subcore_barrier()

    base = c * 5000 + jnp.minimum(s * 320, 4680)
    pltpu.sync_copy(acc_sh.at[pl.ds(base, 320)], deg_hbm.at[pl.ds(base, 320)])


_GB = 80
_NB = _EPT // _GB
_HALF = _N // 2
_HROWS = 5120
_TRASH = 5000


@functools.partial(
    pl.kernel,
    out_type=jax.ShapeDtypeStruct((_N, _D), jnp.float32),
    mesh=_sc_mesh,
    scratch_types=[
        pltpu.VMEM((_NB, _GB), jnp.int32),
        pltpu.VMEM((_NB, _GB), jnp.int32),
        pltpu.VMEM((_GB, _D), jnp.float32),
        pltpu.VMEM_SHARED((_HROWS, _D), jnp.float32),
        pltpu.SemaphoreType.DMA,
    ],
    compiler_params=pltpu.CompilerParams(use_tc_tiling_on_sc=False),
)
def _scatter_sc(src_hbm, lidx_hbm, y_hbm, acc_hbm,
                gidx_v, lidx_v, gbuf, acc_sh, sem):
    c = lax.axis_index("c")
    s = lax.axis_index("s")

    def zrow(i, _):
        for j in range(_D // _L):
            gbuf[i, pl.ds(j * _L, _L)] = jnp.zeros((_L,), jnp.float32)
        return 0

    lax.fori_loop(0, _GB, zrow, 0)
    bz = s * 320
    for r in range(4):
        pltpu.sync_copy(gbuf, acc_sh.at[pl.ds(bz + r * _GB, _GB)])
    plsc.subcore_barrier()

    pltpu.sync_copy(src_hbm.at[pl.ds(s * _NB, _NB)], gidx_v)
    pltpu.sync_copy(lidx_hbm.at[c].at[pl.ds(s * _NB, _NB)], lidx_v)

    def gs(j, _):
        pltpu.async_copy(y_hbm.at[gidx_v.at[j]], gbuf, sem).wait()
        pltpu.sync_copy(gbuf, acc_sh.at[lidx_v.at[j]], add=True)
        return 0

    lax.fori_loop(0, _NB, gs, 0)
    plsc.subcore_barrier()

    bl = jnp.minimum(s * 320, 4680)
    pltpu.sync_copy(acc_sh.at[pl.ds(bl, 320)],
                    acc_hbm.at[pl.ds(c * _HALF + bl, 320)])


def _eidx_kernel(dst_ref, l0_ref, l1_ref):
    d = dst_ref[...]
    in0 = d < _HALF
    l0_ref[...] = jnp.where(in0, d, _TRASH)
    l1_ref[...] = jnp.where(in0, _TRASH, d - _HALF)


def _eidx(dst):
    d2 = dst.reshape(1250, 128)
    l0, l1 = pl.pallas_call(
        _eidx_kernel,
        out_shape=(jax.ShapeDtypeStruct((1250, 128), jnp.int32),
                   jax.ShapeDtypeStruct((1250, 128), jnp.int32)),
    )(d2)
    return jnp.stack([l0.reshape(_NS * _NB, _GB), l1.reshape(_NS * _NB, _GB)])


def _prep_kernel(x_ref, w_ref, deg_ref, y_ref):
    xw = jnp.dot(x_ref[...], w_ref[...], preferred_element_type=jnp.float32)
    y_ref[...] = xw * lax.rsqrt(deg_ref[...] + 1.0)


def _prep(x, w, deg):
    return pl.pallas_call(
        _prep_kernel,
        grid=(_N // _BN,),
        in_specs=[
            pl.BlockSpec((_BN, _D), lambda i: (i, 0)),
            pl.BlockSpec((_D, _D), lambda i: (0, 0)),
            pl.BlockSpec((_BN, 1), lambda i: (i, 0)),
        ],
        out_specs=pl.BlockSpec((_BN, _D), lambda i: (i, 0)),
        out_shape=jax.ShapeDtypeStruct((_N, _D), jnp.float32),
    )(x, w, deg)


def _finalize_kernel(acc_ref, y_ref, deg_ref, b_ref, o_ref):
    dis = lax.rsqrt(deg_ref[...] + 1.0)
    o_ref[...] = jnp.maximum(
        dis * (acc_ref[...] + y_ref[...]) + b_ref[...], 0.0)


def _finalize(acc, y, deg, b):
    return pl.pallas_call(
        _finalize_kernel,
        grid=(_N // _BN,),
        in_specs=[
            pl.BlockSpec((_BN, _D), lambda i: (i, 0)),
            pl.BlockSpec((_BN, _D), lambda i: (i, 0)),
            pl.BlockSpec((_BN, 1), lambda i: (i, 0)),
            pl.BlockSpec((1, _D), lambda i: (0, 0)),
        ],
        out_specs=pl.BlockSpec((_BN, _D), lambda i: (i, 0)),
        out_shape=jax.ShapeDtypeStruct((_N, _D), jnp.float32),
    )(acc, y, deg, b.reshape(1, _D))


def kernel(x, edge_index, W1, b1, W2, b2):
    src = edge_index[0]
    dst = edge_index[1]
    src2d = src.reshape(_NS * _NB, _GB)

    lidx = _eidx(dst)
    deg16 = _deg_sc(dst)
    deg = deg16[:, :1]

    y1 = _prep(x, W1, deg)
    acc1 = _scatter_sc(src2d, lidx, y1)
    h1 = _finalize(acc1, y1, deg, b1)

    y2 = _prep(h1, W2, deg)
    acc2 = _scatter_sc(src2d, lidx, y2)
    return _finalize(acc2, y2, deg, b2)


# speedup vs baseline: 10.0106x; 1.3363x over previous
import functools

import jax
import jax.numpy as jnp
from jax import lax
from jax.experimental import pallas as pl
from jax.experimental.pallas import tpu as pltpu
from jax.experimental.pallas import tpu_sc as plsc

_N, _E, _D = 10000, 160000, 256
_BN = 400

_NC, _NS, _L = 2, 16, 16

_sc_mesh = plsc.VectorSubcoreMesh(core_axis_name="c", subcore_axis_name="s")

_EPT = _E // _NS
_DEGB = 2000
_DEGNB = _EPT // _DEGB


@functools.partial(
    pl.kernel,
    out_type=jax.ShapeDtypeStruct((_N, _L), jnp.float32),
    mesh=_sc_mesh,
    scratch_types=[
        pltpu.VMEM((_DEGB,), jnp.int32),
        pltpu.VMEM((_DEGB, _L), jnp.float32),
        pltpu.VMEM((640, _L), jnp.float32),
        pltpu.VMEM_SHARED((10240, _L), jnp.float32),
    ],
    compiler_params=pltpu.CompilerParams(use_tc_tiling_on_sc=False),
)
def _deg_sc(dst_hbm, deg_hbm, idx_v, ones_v, zeros_v, acc_sh):
    c = lax.axis_index("c")
    s = lax.axis_index("s")

    def fill_ones(i, _):
        ones_v[i] = jnp.ones((_L,), jnp.float32)
        return 0

    lax.fori_loop(0, _DEGB, fill_ones, 0)

    def fill_zeros(i, _):
        zeros_v[i] = jnp.zeros((_L,), jnp.float32)
        return 0

    lax.fori_loop(0, 640, fill_zeros, 0)

    pltpu.sync_copy(zeros_v, acc_sh.at[pl.ds(s * 640, 640)])
    plsc.subcore_barrier()

    def batch(b, _):
        pltpu.sync_copy(dst_hbm.at[pl.ds(s * _EPT + b * _DEGB, _DEGB)], idx_v)
        pltpu.sync_copy(ones_v, acc_sh.at[idx_v], add=True)
        return 0

    lax.fori_loop(0, _DEGNB, batch, 0)
    plsc.subcore_barrier()

    base = c * 5000 + jnp.minimum(s * 320, 4680)
    pltpu.sync_copy(acc_sh.at[pl.ds(base, 320)], deg_hbm.at[pl.ds(base, 320)])


_GB = 50
_NB = _EPT // _GB
_HALF = _N // 2
_HROWS = 5120
_TRASH = 5000


@functools.partial(
    pl.kernel,
    out_type=jax.ShapeDtypeStruct((_N, _D), jnp.float32),
    mesh=_sc_mesh,
    scratch_types=[
        pltpu.VMEM((_NB, _GB), jnp.int32),
        pltpu.VMEM((_NB, _GB), jnp.int32),
        pltpu.VMEM((_GB, _D), jnp.float32),
        pltpu.VMEM((_GB, _D), jnp.float32),
        pltpu.VMEM_SHARED((_HROWS, _D), jnp.float32),
        pltpu.SemaphoreType.DMA,
        pltpu.SemaphoreType.DMA,
    ],
    compiler_params=pltpu.CompilerParams(use_tc_tiling_on_sc=False),
)
def _scatter_sc(src_hbm, lidx_hbm, y_hbm, acc_hbm,
                gidx_v, lidx_v, gbuf0, gbuf1, acc_sh, sem0, sem1):
    c = lax.axis_index("c")
    s = lax.axis_index("s")

    def zrow(i, _):
        for j in range(_D // _L):
            gbuf0[i, pl.ds(j * _L, _L)] = jnp.zeros((_L,), jnp.float32)
        return 0

    lax.fori_loop(0, _GB, zrow, 0)
    bz = s * 320
    for r in range(6):
        pltpu.sync_copy(gbuf0, acc_sh.at[pl.ds(bz + r * _GB, _GB)])
    pltpu.sync_copy(gbuf0.at[pl.ds(0, 20)], acc_sh.at[pl.ds(bz + 300, 20)])
    plsc.subcore_barrier()

    pltpu.sync_copy(src_hbm.at[pl.ds(s * _NB, _NB)], gidx_v)
    pltpu.sync_copy(lidx_hbm.at[c].at[pl.ds(s * _NB, _NB)], lidx_v)

    pltpu.async_copy(y_hbm.at[gidx_v.at[0]], gbuf0, sem0)
    pltpu.async_copy(y_hbm.at[gidx_v.at[1]], gbuf1, sem1)

    def gs2(k, _):
        j0 = 2 * k
        j1 = j0 + 1
        pltpu.make_async_copy(y_hbm.at[gidx_v.at[j0]], gbuf0, sem0).wait()
        pltpu.sync_copy(gbuf0, acc_sh.at[lidx_v.at[j0]], add=True)
        pltpu.async_copy(
            y_hbm.at[gidx_v.at[jnp.minimum(j0 + 2, _NB - 1)]], gbuf0, sem0)
        pltpu.make_async_copy(y_hbm.at[gidx_v.at[j1]], gbuf1, sem1).wait()
        pltpu.sync_copy(gbuf1, acc_sh.at[lidx_v.at[j1]], add=True)
        pltpu.async_copy(
            y_hbm.at[gidx_v.at[jnp.minimum(j1 + 2, _NB - 1)]], gbuf1, sem1)
        return 0

    lax.fori_loop(0, _NB // 2, gs2, 0)
    pltpu.make_async_copy(y_hbm.at[gidx_v.at[_NB - 1]], gbuf0, sem0).wait()
    pltpu.make_async_copy(y_hbm.at[gidx_v.at[_NB - 1]], gbuf1, sem1).wait()
    plsc.subcore_barrier()

    bl = jnp.minimum(s * 320, 4680)
    pltpu.sync_copy(acc_sh.at[pl.ds(bl, 320)],
                    acc_hbm.at[pl.ds(c * _HALF + bl, 320)])


def _eidx_kernel(dst_ref, l0_ref, l1_ref):
    d = dst_ref[...]
    in0 = d < _HALF
    l0_ref[...] = jnp.where(in0, d, _TRASH)
    l1_ref[...] = jnp.where(in0, _TRASH, d - _HALF)


def _eidx(dst):
    d2 = dst.reshape(1250, 128)
    l0, l1 = pl.pallas_call(
        _eidx_kernel,
        out_shape=(jax.ShapeDtypeStruct((1250, 128), jnp.int32),
                   jax.ShapeDtypeStruct((1250, 128), jnp.int32)),
    )(d2)
    return jnp.stack([l0.reshape(_NS * _NB, _GB), l1.reshape(_NS * _NB, _GB)])


def _prep_kernel(x_ref, w_ref, deg_ref, y_ref):
    xw = jnp.dot(x_ref[...], w_ref[...], preferred_element_type=jnp.float32)
    y_ref[...] = xw * lax.rsqrt(deg_ref[...] + 1.0)


def _prep(x, w, deg):
    return pl.pallas_call(
        _prep_kernel,
        grid=(_N // _BN,),
        in_specs=[
            pl.BlockSpec((_BN, _D), lambda i: (i, 0)),
            pl.BlockSpec((_D, _D), lambda i: (0, 0)),
            pl.BlockSpec((_BN, 1), lambda i: (i, 0)),
        ],
        out_specs=pl.BlockSpec((_BN, _D), lambda i: (i, 0)),
        out_shape=jax.ShapeDtypeStruct((_N, _D), jnp.float32),
    )(x, w, deg)


def _finalize_kernel(acc_ref, y_ref, deg_ref, b_ref, o_ref):
    dis = lax.rsqrt(deg_ref[...] + 1.0)
    o_ref[...] = jnp.maximum(
        dis * (acc_ref[...] + y_ref[...]) + b_ref[...], 0.0)


def _finalize(acc, y, deg, b):
    return pl.pallas_call(
        _finalize_kernel,
        grid=(_N // _BN,),
        in_specs=[
            pl.BlockSpec((_BN, _D), lambda i: (i, 0)),
            pl.BlockSpec((_BN, _D), lambda i: (i, 0)),
            pl.BlockSpec((_BN, 1), lambda i: (i, 0)),
            pl.BlockSpec((1, _D), lambda i: (0, 0)),
        ],
        out_specs=pl.BlockSpec((_BN, _D), lambda i: (i, 0)),
        out_shape=jax.ShapeDtypeStruct((_N, _D), jnp.float32),
    )(acc, y, deg, b.reshape(1, _D))


def kernel(x, edge_index, W1, b1, W2, b2):
    src = edge_index[0]
    dst = edge_index[1]
    src2d = src.reshape(_NS * _NB, _GB)

    lidx = _eidx(dst)
    deg16 = _deg_sc(dst)
    deg = deg16[:, :1]

    y1 = _prep(x, W1, deg)
    acc1 = _scatter_sc(src2d, lidx, y1)
    h1 = _finalize(acc1, y1, deg, b1)

    y2 = _prep(h1, W2, deg)
    acc2 = _scatter_sc(src2d, lidx, y2)
    return _finalize(acc2, y2, deg, b2)


# speedup vs baseline: 10.1871x; 1.0176x over previous
import functools

import jax
import jax.numpy as jnp
from jax import lax
from jax.experimental import pallas as pl
from jax.experimental.pallas import tpu as pltpu
from jax.experimental.pallas import tpu_sc as plsc

_N, _E, _D = 10000, 160000, 256
_BN = 400

_NC, _NS, _L = 2, 16, 16

_sc_mesh = plsc.VectorSubcoreMesh(core_axis_name="c", subcore_axis_name="s")

_EPT = _E // _NS
_DEGB = 2000
_DEGNB = _EPT // _DEGB


@functools.partial(
    pl.kernel,
    out_type=jax.ShapeDtypeStruct((_N, _L), jnp.float32),
    mesh=_sc_mesh,
    scratch_types=[
        pltpu.VMEM((_DEGB,), jnp.int32),
        pltpu.VMEM((_DEGB, _L), jnp.float32),
        pltpu.VMEM((640, _L), jnp.float32),
        pltpu.VMEM_SHARED((10240, _L), jnp.float32),
    ],
    compiler_params=pltpu.CompilerParams(use_tc_tiling_on_sc=False),
)
def _deg_sc(dst_hbm, deg_hbm, idx_v, ones_v, zeros_v, acc_sh):
    c = lax.axis_index("c")
    s = lax.axis_index("s")

    def fill_ones(i, _):
        ones_v[i] = jnp.ones((_L,), jnp.float32)
        return 0

    lax.fori_loop(0, _DEGB, fill_ones, 0)

    def fill_zeros(i, _):
        zeros_v[i] = jnp.zeros((_L,), jnp.float32)
        return 0

    lax.fori_loop(0, 640, fill_zeros, 0)

    pltpu.sync_copy(zeros_v, acc_sh.at[pl.ds(s * 640, 640)])
    plsc.subcore_barrier()

    def batch(b, _):
        pltpu.sync_copy(dst_hbm.at[pl.ds(s * _EPT + b * _DEGB, _DEGB)], idx_v)
        pltpu.sync_copy(ones_v, acc_sh.at[idx_v], add=True)
        return 0

    lax.fori_loop(0, _DEGNB, batch, 0)
    plsc.subcore_barrier()

    base = c * 5000 + jnp.minimum(s * 320, 4680)
    pltpu.sync_copy(acc_sh.at[pl.ds(base, 320)], deg_hbm.at[pl.ds(base, 320)])


_GB = 50
_NB = _EPT // _GB
_HALF = _N // 2
_HROWS = 5120
_TRASH = 5000


@functools.partial(
    pl.kernel,
    out_type=jax.ShapeDtypeStruct((_N, _D), jnp.float32),
    mesh=_sc_mesh,
    scratch_types=[
        pltpu.VMEM((_NB, _GB), jnp.int32),
        pltpu.VMEM((_NB, _GB), jnp.int32),
        pltpu.VMEM((_GB, _D), jnp.float32),
        pltpu.VMEM((_GB, _D), jnp.float32),
        pltpu.VMEM_SHARED((_HROWS, _D), jnp.float32),
        pltpu.SemaphoreType.DMA,
        pltpu.SemaphoreType.DMA,
    ],
    compiler_params=pltpu.CompilerParams(use_tc_tiling_on_sc=False),
)
def _scatter_sc(src_hbm, lidx_hbm, y_hbm, acc_hbm,
                gidx_v, lidx_v, gbuf0, gbuf1, acc_sh, sem0, sem1):
    c = lax.axis_index("c")
    s = lax.axis_index("s")

    def zrow(i, _):
        for j in range(_D // _L):
            gbuf0[i, pl.ds(j * _L, _L)] = jnp.zeros((_L,), jnp.float32)
        return 0

    lax.fori_loop(0, _GB, zrow, 0)
    bz = s * 320
    for r in range(6):
        pltpu.sync_copy(gbuf0, acc_sh.at[pl.ds(bz + r * _GB, _GB)])
    pltpu.sync_copy(gbuf0.at[pl.ds(0, 20)], acc_sh.at[pl.ds(bz + 300, 20)])
    plsc.subcore_barrier()

    pltpu.sync_copy(src_hbm.at[pl.ds(s * _NB, _NB)], gidx_v)
    pltpu.sync_copy(lidx_hbm.at[c].at[pl.ds(s * _NB, _NB)], lidx_v)

    pltpu.async_copy(y_hbm.at[gidx_v.at[0]], gbuf0, sem0)
    pltpu.async_copy(y_hbm.at[gidx_v.at[1]], gbuf1, sem1)

    def gs2(k, _):
        j0 = 2 * k
        j1 = j0 + 1
        pltpu.make_async_copy(y_hbm.at[gidx_v.at[j0]], gbuf0, sem0).wait()
        pltpu.sync_copy(gbuf0, acc_sh.at[lidx_v.at[j0]], add=True)
        pltpu.async_copy(
            y_hbm.at[gidx_v.at[jnp.minimum(j0 + 2, _NB - 1)]], gbuf0, sem0)
        pltpu.make_async_copy(y_hbm.at[gidx_v.at[j1]], gbuf1, sem1).wait()
        pltpu.sync_copy(gbuf1, acc_sh.at[lidx_v.at[j1]], add=True)
        pltpu.async_copy(
            y_hbm.at[gidx_v.at[jnp.minimum(j1 + 2, _NB - 1)]], gbuf1, sem1)
        return 0

    lax.fori_loop(0, _NB // 2, gs2, 0)
    pltpu.make_async_copy(y_hbm.at[gidx_v.at[_NB - 1]], gbuf0, sem0).wait()
    pltpu.make_async_copy(y_hbm.at[gidx_v.at[_NB - 1]], gbuf1, sem1).wait()
    plsc.subcore_barrier()

    bl = jnp.minimum(s * 320, 4680)
    pltpu.sync_copy(acc_sh.at[pl.ds(bl, 320)],
                    acc_hbm.at[pl.ds(c * _HALF + bl, 320)])


def _eidx_kernel(dst_ref, l_ref):
    d = dst_ref[...]
    in0 = d < _HALF
    l_ref[0] = jnp.where(in0, d, _TRASH)
    l_ref[1] = jnp.where(in0, _TRASH, d - _HALF)


def _eidx(dst):
    d2 = dst.reshape(1250, 128)
    l = pl.pallas_call(
        _eidx_kernel,
        out_shape=jax.ShapeDtypeStruct((2, 1250, 128), jnp.int32),
    )(d2)
    return l.reshape(2, _NS * _NB, _GB)


def _prep_kernel(x_ref, w_ref, deg_ref, y_ref):
    xw = jnp.dot(x_ref[...], w_ref[...], preferred_element_type=jnp.float32)
    y_ref[...] = xw * lax.rsqrt(deg_ref[:, :1] + 1.0)


def _prep(x, w, deg16):
    return pl.pallas_call(
        _prep_kernel,
        grid=(_N // _BN,),
        in_specs=[
            pl.BlockSpec((_BN, _D), lambda i: (i, 0)),
            pl.BlockSpec((_D, _D), lambda i: (0, 0)),
            pl.BlockSpec((_BN, _L), lambda i: (i, 0)),
        ],
        out_specs=pl.BlockSpec((_BN, _D), lambda i: (i, 0)),
        out_shape=jax.ShapeDtypeStruct((_N, _D), jnp.float32),
    )(x, w, deg16)


def _mid_kernel(acc_ref, y_ref, deg_ref, b_ref, w_ref, y2_ref):
    dis = lax.rsqrt(deg_ref[:, :1] + 1.0)
    h = jnp.maximum(dis * (acc_ref[...] + y_ref[...]) + b_ref[...], 0.0)
    xw = jnp.dot(h, w_ref[...], preferred_element_type=jnp.float32)
    y2_ref[...] = xw * dis


def _mid(acc, y, deg16, b, w):
    return pl.pallas_call(
        _mid_kernel,
        grid=(_N // _BN,),
        in_specs=[
            pl.BlockSpec((_BN, _D), lambda i: (i, 0)),
            pl.BlockSpec((_BN, _D), lambda i: (i, 0)),
            pl.BlockSpec((_BN, _L), lambda i: (i, 0)),
            pl.BlockSpec((1, _D), lambda i: (0, 0)),
            pl.BlockSpec((_D, _D), lambda i: (0, 0)),
        ],
        out_specs=pl.BlockSpec((_BN, _D), lambda i: (i, 0)),
        out_shape=jax.ShapeDtypeStruct((_N, _D), jnp.float32),
    )(acc, y, deg16, b.reshape(1, _D), w)


def _finalize_kernel(acc_ref, y_ref, deg_ref, b_ref, o_ref):
    dis = lax.rsqrt(deg_ref[:, :1] + 1.0)
    o_ref[...] = jnp.maximum(
        dis * (acc_ref[...] + y_ref[...]) + b_ref[...], 0.0)


def _finalize(acc, y, deg16, b):
    return pl.pallas_call(
        _finalize_kernel,
        grid=(_N // _BN,),
        in_specs=[
            pl.BlockSpec((_BN, _D), lambda i: (i, 0)),
            pl.BlockSpec((_BN, _D), lambda i: (i, 0)),
            pl.BlockSpec((_BN, _L), lambda i: (i, 0)),
            pl.BlockSpec((1, _D), lambda i: (0, 0)),
        ],
        out_specs=pl.BlockSpec((_BN, _D), lambda i: (i, 0)),
        out_shape=jax.ShapeDtypeStruct((_N, _D), jnp.float32),
    )(acc, y, deg16, b.reshape(1, _D))


def kernel(x, edge_index, W1, b1, W2, b2):
    src = edge_index[0]
    dst = edge_index[1]
    src2d = src.reshape(_NS * _NB, _GB)

    lidx = _eidx(dst)
    deg16 = _deg_sc(dst)

    y1 = _prep(x, W1, deg16)
    acc1 = _scatter_sc(src2d, lidx, y1)
    y2 = _mid(acc1, y1, deg16, b1, W2)
    acc2 = _scatter_sc(src2d, lidx, y2)
    return _finalize(acc2, y2, deg16, b2)


# speedup vs baseline: 17.6991x; 1.7374x over previous
import functools

import jax
import jax.numpy as jnp
from jax import lax
from jax.experimental import pallas as pl
from jax.experimental.pallas import tpu as pltpu
from jax.experimental.pallas import tpu_sc as plsc

_N, _E, _D = 10000, 160000, 256
_BN = 400
_DH = _D // 2

_NC, _NS, _L = 2, 16, 16

_sc_mesh = plsc.VectorSubcoreMesh(core_axis_name="c", subcore_axis_name="s")

_EPT = _E // _NS
_DEGB = 2000
_DEGNB = _EPT // _DEGB


@functools.partial(
    pl.kernel,
    out_type=jax.ShapeDtypeStruct((_N, _L), jnp.float32),
    mesh=_sc_mesh,
    scratch_types=[
        pltpu.VMEM((_DEGB,), jnp.int32),
        pltpu.VMEM((_DEGB, _L), jnp.float32),
        pltpu.VMEM((640, _L), jnp.float32),
        pltpu.VMEM_SHARED((10240, _L), jnp.float32),
    ],
    compiler_params=pltpu.CompilerParams(use_tc_tiling_on_sc=False),
)
def _deg_sc(dst_hbm, deg_hbm, idx_v, ones_v, zeros_v, acc_sh):
    c = lax.axis_index("c")
    s = lax.axis_index("s")

    def fill_ones(i, _):
        ones_v[i] = jnp.ones((_L,), jnp.float32)
        return 0

    lax.fori_loop(0, _DEGB, fill_ones, 0)

    def fill_zeros(i, _):
        zeros_v[i] = jnp.zeros((_L,), jnp.float32)
        return 0

    lax.fori_loop(0, 640, fill_zeros, 0)

    pltpu.sync_copy(zeros_v, acc_sh.at[pl.ds(s * 640, 640)])
    plsc.subcore_barrier()

    def batch(b, _):
        pltpu.sync_copy(dst_hbm.at[pl.ds(s * _EPT + b * _DEGB, _DEGB)], idx_v)
        pltpu.sync_copy(ones_v, acc_sh.at[idx_v], add=True)
        return 0

    lax.fori_loop(0, _DEGNB, batch, 0)
    plsc.subcore_barrier()

    base = c * 5000 + jnp.minimum(s * 320, 4680)
    pltpu.sync_copy(acc_sh.at[pl.ds(base, 320)], deg_hbm.at[pl.ds(base, 320)])


_GB = 100
_NB = _EPT // _GB
_AROWS = 10016


@functools.partial(
    pl.kernel,
    out_type=jax.ShapeDtypeStruct((_NC, _N, _DH), jnp.float32),
    mesh=_sc_mesh,
    scratch_types=[
        pltpu.VMEM((_NB, _GB), jnp.int32),
        pltpu.VMEM((_NB, _GB), jnp.int32),
        pltpu.VMEM((_GB, _DH), jnp.float32),
        pltpu.VMEM((_GB, _DH), jnp.float32),
        pltpu.VMEM_SHARED((_AROWS, _DH), jnp.float32),
        pltpu.SemaphoreType.DMA,
        pltpu.SemaphoreType.DMA,
    ],
    compiler_params=pltpu.CompilerParams(use_tc_tiling_on_sc=False),
)
def _scatter_sc(src_hbm, dst_hbm, y_hbm, acc_hbm,
                gidx_v, didx_v, gbuf0, gbuf1, acc_sh, sem0, sem1):
    c = lax.axis_index("c")
    s = lax.axis_index("s")

    def zrow(i, _):
        for j in range(_DH // _L):
            gbuf0[i, pl.ds(j * _L, _L)] = jnp.zeros((_L,), jnp.float32)
        return 0

    lax.fori_loop(0, _GB, zrow, 0)
    bz = jnp.minimum(s * 640, _AROWS - 640)
    for r in range(6):
        pltpu.sync_copy(gbuf0, acc_sh.at[pl.ds(bz + r * _GB, _GB)])
    pltpu.sync_copy(gbuf0.at[pl.ds(0, 40)], acc_sh.at[pl.ds(bz + 600, 40)])
    plsc.subcore_barrier()

    pltpu.sync_copy(src_hbm.at[pl.ds(s * _NB, _NB)], gidx_v)
    pltpu.sync_copy(dst_hbm.at[pl.ds(s * _NB, _NB)], didx_v)

    yc = y_hbm.at[c]
    pltpu.async_copy(yc.at[gidx_v.at[0]], gbuf0, sem0)
    pltpu.async_copy(yc.at[gidx_v.at[1]], gbuf1, sem1)

    def gs2(k, _):
        j0 = 2 * k
        j1 = j0 + 1
        pltpu.make_async_copy(yc.at[gidx_v.at[j0]], gbuf0, sem0).wait()
        pltpu.sync_copy(gbuf0, acc_sh.at[didx_v.at[j0]], add=True)
        pltpu.async_copy(
            yc.at[gidx_v.at[jnp.minimum(j0 + 2, _NB - 1)]], gbuf0, sem0)
        pltpu.make_async_copy(yc.at[gidx_v.at[j1]], gbuf1, sem1).wait()
        pltpu.sync_copy(gbuf1, acc_sh.at[didx_v.at[j1]], add=True)
        pltpu.async_copy(
            yc.at[gidx_v.at[jnp.minimum(j1 + 2, _NB - 1)]], gbuf1, sem1)
        return 0

    lax.fori_loop(0, _NB // 2, gs2, 0)
    pltpu.make_async_copy(yc.at[gidx_v.at[_NB - 1]], gbuf0, sem0).wait()
    pltpu.make_async_copy(yc.at[gidx_v.at[_NB - 1]], gbuf1, sem1).wait()
    plsc.subcore_barrier()

    bl = jnp.minimum(s * 640, _N - 640)
    pltpu.sync_copy(acc_sh.at[pl.ds(bl, 640)],
                    acc_hbm.at[c].at[pl.ds(bl, 640)])


def _prep_kernel(x_ref, w_ref, deg_ref, y_ref):
    xw = jnp.dot(x_ref[...], w_ref[...], preferred_element_type=jnp.float32)
    y = xw * lax.rsqrt(deg_ref[:, :1] + 1.0)
    y_ref[0] = y[:, :_DH]
    y_ref[1] = y[:, _DH:]


def _prep(x, w, deg16):
    return pl.pallas_call(
        _prep_kernel,
        grid=(_N // _BN,),
        in_specs=[
            pl.BlockSpec((_BN, _D), lambda i: (i, 0)),
            pl.BlockSpec((_D, _D), lambda i: (0, 0)),
            pl.BlockSpec((_BN, _L), lambda i: (i, 0)),
        ],
        out_specs=pl.BlockSpec((_NC, _BN, _DH), lambda i: (0, i, 0)),
        out_shape=jax.ShapeDtypeStruct((_NC, _N, _DH), jnp.float32),
    )(x, w, deg16)


def _mid_kernel(acc_ref, y_ref, deg_ref, b_ref, w_ref, y2_ref):
    dis = lax.rsqrt(deg_ref[:, :1] + 1.0)
    acc = jnp.concatenate([acc_ref[0], acc_ref[1]], axis=1)
    y = jnp.concatenate([y_ref[0], y_ref[1]], axis=1)
    h = jnp.maximum(dis * (acc + y) + b_ref[...], 0.0)
    xw = jnp.dot(h, w_ref[...], preferred_element_type=jnp.float32)
    y2 = xw * dis
    y2_ref[0] = y2[:, :_DH]
    y2_ref[1] = y2[:, _DH:]


def _mid(acc, y, deg16, b, w):
    return pl.pallas_call(
        _mid_kernel,
        grid=(_N // _BN,),
        in_specs=[
            pl.BlockSpec((_NC, _BN, _DH), lambda i: (0, i, 0)),
            pl.BlockSpec((_NC, _BN, _DH), lambda i: (0, i, 0)),
            pl.BlockSpec((_BN, _L), lambda i: (i, 0)),
            pl.BlockSpec((1, _D), lambda i: (0, 0)),
            pl.BlockSpec((_D, _D), lambda i: (0, 0)),
        ],
        out_specs=pl.BlockSpec((_NC, _BN, _DH), lambda i: (0, i, 0)),
        out_shape=jax.ShapeDtypeStruct((_NC, _N, _DH), jnp.float32),
    )(acc, y, deg16, b.reshape(1, _D), w)


def _finalize_kernel(acc_ref, y_ref, deg_ref, b_ref, o_ref):
    dis = lax.rsqrt(deg_ref[:, :1] + 1.0)
    acc = jnp.concatenate([acc_ref[0], acc_ref[1]], axis=1)
    y = jnp.concatenate([y_ref[0], y_ref[1]], axis=1)
    o_ref[...] = jnp.maximum(dis * (acc + y) + b_ref[...], 0.0)


def _finalize(acc, y, deg16, b):
    return pl.pallas_call(
        _finalize_kernel,
        grid=(_N // _BN,),
        in_specs=[
            pl.BlockSpec((_NC, _BN, _DH), lambda i: (0, i, 0)),
            pl.BlockSpec((_NC, _BN, _DH), lambda i: (0, i, 0)),
            pl.BlockSpec((_BN, _L), lambda i: (i, 0)),
            pl.BlockSpec((1, _D), lambda i: (0, 0)),
        ],
        out_specs=pl.BlockSpec((_BN, _D), lambda i: (i, 0)),
        out_shape=jax.ShapeDtypeStruct((_N, _D), jnp.float32),
    )(acc, y, deg16, b.reshape(1, _D))


def kernel(x, edge_index, W1, b1, W2, b2):
    src = edge_index[0]
    dst = edge_index[1]
    src2d = src.reshape(_NS * _NB, _GB)
    dst2d = dst.reshape(_NS * _NB, _GB)

    deg16 = _deg_sc(dst)

    y1 = _prep(x, W1, deg16)
    acc1 = _scatter_sc(src2d, dst2d, y1)
    y2 = _mid(acc1, y1, deg16, b1, W2)
    acc2 = _scatter_sc(src2d, dst2d, y2)
    return _finalize(acc2, y2, deg16, b2)


# speedup vs baseline: 19.6303x; 1.1091x over previous
import functools

import jax
import jax.numpy as jnp
from jax import lax
from jax.experimental import pallas as pl
from jax.experimental.pallas import tpu as pltpu
from jax.experimental.pallas import tpu_sc as plsc

_N, _E, _D = 10000, 160000, 256
_BN = 400
_DH = _D // 2

_NC, _NS, _L = 2, 16, 16

_sc_mesh = plsc.VectorSubcoreMesh(core_axis_name="c", subcore_axis_name="s")

_EPT = _E // _NS
_DEGB = 2000
_DEGNB = _EPT // _DEGB


@functools.partial(
    pl.kernel,
    out_type=jax.ShapeDtypeStruct((_N, _L), jnp.float32),
    mesh=_sc_mesh,
    scratch_types=[
        pltpu.VMEM((_DEGB,), jnp.int32),
        pltpu.VMEM((_DEGB, _L), jnp.float32),
        pltpu.VMEM((640, _L), jnp.float32),
        pltpu.VMEM_SHARED((10240, _L), jnp.float32),
    ],
    compiler_params=pltpu.CompilerParams(use_tc_tiling_on_sc=False),
)
def _deg_sc(dst_hbm, deg_hbm, idx_v, ones_v, zeros_v, acc_sh):
    c = lax.axis_index("c")
    s = lax.axis_index("s")

    def fill_ones(i, _):
        ones_v[i] = jnp.ones((_L,), jnp.float32)
        return 0

    lax.fori_loop(0, _DEGB, fill_ones, 0)

    def fill_zeros(i, _):
        zeros_v[i] = jnp.zeros((_L,), jnp.float32)
        return 0

    lax.fori_loop(0, 640, fill_zeros, 0)

    pltpu.sync_copy(zeros_v, acc_sh.at[pl.ds(s * 640, 640)])
    plsc.subcore_barrier()

    def batch(b, _):
        pltpu.sync_copy(dst_hbm.at[pl.ds(s * _EPT + b * _DEGB, _DEGB)], idx_v)
        pltpu.sync_copy(ones_v, acc_sh.at[idx_v], add=True)
        return 0

    lax.fori_loop(0, _DEGNB, batch, 0)
    plsc.subcore_barrier()

    base = c * 5000 + jnp.minimum(s * 320, 4680)
    pltpu.sync_copy(acc_sh.at[pl.ds(base, 320)], deg_hbm.at[pl.ds(base, 320)])


_GB = 50
_NB = _EPT // _GB
_AROWS = 10016


@functools.partial(
    pl.kernel,
    out_type=jax.ShapeDtypeStruct((_NC, _N, _DH), jnp.float32),
    mesh=_sc_mesh,
    scratch_types=[
        pltpu.VMEM((_NB, _GB), jnp.int32),
        pltpu.VMEM((_NB, _GB), jnp.int32),
        pltpu.VMEM((_GB, _DH), jnp.float32),
        pltpu.VMEM((_GB, _DH), jnp.float32),
        pltpu.VMEM((_GB, _DH), jnp.float32),
        pltpu.VMEM((_GB, _DH), jnp.float32),
        pltpu.VMEM_SHARED((_AROWS, _DH), jnp.float32),
        pltpu.SemaphoreType.DMA,
        pltpu.SemaphoreType.DMA,
        pltpu.SemaphoreType.DMA,
        pltpu.SemaphoreType.DMA,
    ],
    compiler_params=pltpu.CompilerParams(use_tc_tiling_on_sc=False),
)
def _scatter_sc(src_hbm, dst_hbm, y_hbm, acc_hbm,
                gidx_v, didx_v, gbuf0, gbuf1, gbuf2, gbuf3, acc_sh,
                sem0, sem1, sem2, sem3):
    c = lax.axis_index("c")
    s = lax.axis_index("s")

    def zrow(i, _):
        for j in range(_DH // _L):
            gbuf0[i, pl.ds(j * _L, _L)] = jnp.zeros((_L,), jnp.float32)
        return 0

    lax.fori_loop(0, _GB, zrow, 0)
    bz = jnp.minimum(s * 640, _AROWS - 640)
    for r in range(12):
        pltpu.sync_copy(gbuf0, acc_sh.at[pl.ds(bz + r * _GB, _GB)])
    pltpu.sync_copy(gbuf0.at[pl.ds(0, 40)], acc_sh.at[pl.ds(bz + 600, 40)])
    plsc.subcore_barrier()

    pltpu.sync_copy(src_hbm.at[pl.ds(s * _NB, _NB)], gidx_v)
    pltpu.sync_copy(dst_hbm.at[pl.ds(s * _NB, _NB)], didx_v)

    yc = y_hbm.at[c]
    bufs = (gbuf0, gbuf1, gbuf2, gbuf3)
    sems = (sem0, sem1, sem2, sem3)
    for r in range(4):
        pltpu.async_copy(yc.at[gidx_v.at[r]], bufs[r], sems[r])

    def gs4(k, _):
        for r in range(4):
            j = 4 * k + r
            pltpu.make_async_copy(yc.at[gidx_v.at[j]], bufs[r], sems[r]).wait()
            pltpu.sync_copy(bufs[r], acc_sh.at[didx_v.at[j]], add=True)
            pltpu.async_copy(
                yc.at[gidx_v.at[jnp.minimum(j + 4, _NB - 1)]], bufs[r], sems[r])
        return 0

    lax.fori_loop(0, _NB // 4, gs4, 0)
    for r in range(4):
        pltpu.make_async_copy(yc.at[gidx_v.at[_NB - 1]], bufs[r], sems[r]).wait()
    plsc.subcore_barrier()

    bl = jnp.minimum(s * 640, _N - 640)
    pltpu.sync_copy(acc_sh.at[pl.ds(bl, 640)],
                    acc_hbm.at[c].at[pl.ds(bl, 640)])


def _prep_kernel(x_ref, w_ref, deg_ref, y_ref):
    xw = jnp.dot(x_ref[...], w_ref[...], preferred_element_type=jnp.float32)
    y = xw * lax.rsqrt(deg_ref[:, :1] + 1.0)
    y_ref[0] = y[:, :_DH]
    y_ref[1] = y[:, _DH:]


def _prep(x, w, deg16):
    return pl.pallas_call(
        _prep_kernel,
        grid=(_N // _BN,),
        in_specs=[
            pl.BlockSpec((_BN, _D), lambda i: (i, 0)),
            pl.BlockSpec((_D, _D), lambda i: (0, 0)),
            pl.BlockSpec((_BN, _L), lambda i: (i, 0)),
        ],
        out_specs=pl.BlockSpec((_NC, _BN, _DH), lambda i: (0, i, 0)),
        out_shape=jax.ShapeDtypeStruct((_NC, _N, _DH), jnp.float32),
    )(x, w, deg16)


def _mid_kernel(acc_ref, y_ref, deg_ref, b_ref, w_ref, y2_ref):
    dis = lax.rsqrt(deg_ref[:, :1] + 1.0)
    acc = jnp.concatenate([acc_ref[0], acc_ref[1]], axis=1)
    y = jnp.concatenate([y_ref[0], y_ref[1]], axis=1)
    h = jnp.maximum(dis * (acc + y) + b_ref[...], 0.0)
    xw = jnp.dot(h, w_ref[...], preferred_element_type=jnp.float32)
    y2 = xw * dis
    y2_ref[0] = y2[:, :_DH]
    y2_ref[1] = y2[:, _DH:]


def _mid(acc, y, deg16, b, w):
    return pl.pallas_call(
        _mid_kernel,
        grid=(_N // _BN,),
        in_specs=[
            pl.BlockSpec((_NC, _BN, _DH), lambda i: (0, i, 0)),
            pl.BlockSpec((_NC, _BN, _DH), lambda i: (0, i, 0)),
            pl.BlockSpec((_BN, _L), lambda i: (i, 0)),
            pl.BlockSpec((1, _D), lambda i: (0, 0)),
            pl.BlockSpec((_D, _D), lambda i: (0, 0)),
        ],
        out_specs=pl.BlockSpec((_NC, _BN, _DH), lambda i: (0, i, 0)),
        out_shape=jax.ShapeDtypeStruct((_NC, _N, _DH), jnp.float32),
    )(acc, y, deg16, b.reshape(1, _D), w)


def _finalize_kernel(acc_ref, y_ref, deg_ref, b_ref, o_ref):
    dis = lax.rsqrt(deg_ref[:, :1] + 1.0)
    acc = jnp.concatenate([acc_ref[0], acc_ref[1]], axis=1)
    y = jnp.concatenate([y_ref[0], y_ref[1]], axis=1)
    o_ref[...] = jnp.maximum(dis * (acc + y) + b_ref[...], 0.0)


def _finalize(acc, y, deg16, b):
    return pl.pallas_call(
        _finalize_kernel,
        grid=(_N // _BN,),
        in_specs=[
            pl.BlockSpec((_NC, _BN, _DH), lambda i: (0, i, 0)),
            pl.BlockSpec((_NC, _BN, _DH), lambda i: (0, i, 0)),
            pl.BlockSpec((_BN, _L), lambda i: (i, 0)),
            pl.BlockSpec((1, _D), lambda i: (0, 0)),
        ],
        out_specs=pl.BlockSpec((_BN, _D), lambda i: (i, 0)),
        out_shape=jax.ShapeDtypeStruct((_N, _D), jnp.float32),
    )(acc, y, deg16, b.reshape(1, _D))


def kernel(x, edge_index, W1, b1, W2, b2):
    src = edge_index[0]
    dst = edge_index[1]
    src2d = src.reshape(_NS * _NB, _GB)
    dst2d = dst.reshape(_NS * _NB, _GB)

    deg16 = _deg_sc(dst)

    y1 = _prep(x, W1, deg16)
    acc1 = _scatter_sc(src2d, dst2d, y1)
    y2 = _mid(acc1, y1, deg16, b1, W2)
    acc2 = _scatter_sc(src2d, dst2d, y2)
    return _finalize(acc2, y2, deg16, b2)
